# Initial kernel scaffold; baseline (speedup 1.0000x reference)
import functools

import jax
import jax.numpy as jnp
from jax import lax
from jax.experimental import pallas as pl
from jax.experimental.pallas import tpu as pltpu, tpu_sc as plsc

N_NODES = 100000
N_EDGES = 6400000
NC = 2
NS = 16
NW = NC * NS
CH = 128
N_CHUNKS = N_EDGES // CH
BASE_CHUNKS = N_CHUNKS // NW
REM_CHUNKS = N_CHUNKS % NW
ROWS_PER_TILE = N_NODES // NS


def _make_agg(dp):
    mesh = plsc.VectorSubcoreMesh(
        core_axis_name="c", subcore_axis_name="s", num_cores=NC, num_subcores=NS
    )

    @functools.partial(
        pl.kernel,
        out_type=jax.ShapeDtypeStruct((NC, N_NODES, dp), jnp.float32),
        mesh=mesh,
        scratch_types=[
            pltpu.VMEM((2, CH), jnp.int32),
            pltpu.VMEM((CH, dp), jnp.float32),
            pltpu.VMEM_SHARED((N_NODES, dp), jnp.float32),
        ],
    )
    def agg_kernel(table, edges, zeros, out, idx_v, rows_v, acc):
        cid = lax.axis_index("c")
        sid = lax.axis_index("s")
        wid = sid * NC + cid

        r0 = sid * ROWS_PER_TILE
        pltpu.sync_copy(
            zeros.at[pl.ds(r0, ROWS_PER_TILE)], acc.at[pl.ds(r0, ROWS_PER_TILE)]
        )
        plsc.subcore_barrier()

        n_chunks = jnp.where(wid < REM_CHUNKS, BASE_CHUNKS + 1, BASE_CHUNKS)

        @pl.loop(0, n_chunks)
        def _(j):
            start = (wid + j * NW) * CH
            pltpu.sync_copy(edges.at[:, pl.ds(start, CH)], idx_v)
            pltpu.sync_copy(table.at[idx_v.at[0]], rows_v)
            pltpu.sync_copy(rows_v, acc.at[idx_v.at[1]], add=True)

        plsc.subcore_barrier()
        pltpu.sync_copy(
            acc.at[pl.ds(r0, ROWS_PER_TILE)], out.at[cid, pl.ds(r0, ROWS_PER_TILE)]
        )

    return agg_kernel


_agg8 = _make_agg(8)
_agg16 = _make_agg(16)
_agg20 = _make_agg(20)

_BLK = 2000
_GRID = N_NODES // _BLK


def _l1_body(p_ref, x_ref, wl_ref, bl_ref, wr_ref, h_ref, inv_ref):
    p = p_ref[0] + p_ref[1]
    inv = 1.0 / jnp.maximum(p[:, 4:5], 1.0)
    mean = p[:, :4] * inv
    h = jnp.maximum(mean @ wl_ref[...] + bl_ref[...] + x_ref[...] @ wr_ref[...], 0.0)
    h_ref[...] = jnp.concatenate([h, jnp.zeros((_BLK, 6), jnp.float32)], axis=1)
    inv_ref[...] = inv


def _l2_body(p_ref, x_ref, inv_ref, wl_ref, bl_ref, wr_ref, h_ref):
    p = p_ref[0] + p_ref[1]
    mean = p[:, :10] * inv_ref[...]
    x10 = x_ref[...][:, :10]
    h_ref[...] = jnp.maximum(
        mean @ wl_ref[...] + bl_ref[...] + x10 @ wr_ref[...], 0.0
    )


def _l3_body(p_ref, x_ref, inv_ref, wl_ref, bl_ref, wr_ref, wc_ref, bc_ref, o_ref):
    p = p_ref[0] + p_ref[1]
    mean = p * inv_ref[...]
    h = jnp.maximum(mean @ wl_ref[...] + bl_ref[...] + x_ref[...] @ wr_ref[...], 0.0)
    o_ref[...] = h @ wc_ref[...] + bc_ref[...]


def _whole(shape):
    return pl.BlockSpec(shape, lambda i: (0,) * len(shape))


def _rows(d):
    return pl.BlockSpec((_BLK, d), lambda i: (i, 0))


def _part(dp):
    return pl.BlockSpec((2, _BLK, dp), lambda i: (0, i, 0))


def _dense1(part1, x, wl_t, bl, wr_t):
    return pl.pallas_call(
        _l1_body,
        grid=(_GRID,),
        in_specs=[_part(8), _rows(4), _whole((4, 10)), _whole((10,)), _whole((4, 10))],
        out_specs=[_rows(16), _rows(1)],
        out_shape=[
            jax.ShapeDtypeStruct((N_NODES, 16), jnp.float32),
            jax.ShapeDtypeStruct((N_NODES, 1), jnp.float32),
        ],
    )(part1, x, wl_t, bl, wr_t)


def _dense2(part2, h1p, inv, wl_t, bl, wr_t):
    return pl.pallas_call(
        _l2_body,
        grid=(_GRID,),
        in_specs=[
            _part(16), _rows(16), _rows(1),
            _whole((10, 20)), _whole((20,)), _whole((10, 20)),
        ],
        out_specs=_rows(20),
        out_shape=jax.ShapeDtypeStruct((N_NODES, 20), jnp.float32),
    )(part2, h1p, inv, wl_t, bl, wr_t)


def _dense3(part3, h2, inv, wl_t, bl, wr_t, wc_t, bc):
    return pl.pallas_call(
        _l3_body,
        grid=(_GRID,),
        in_specs=[
            _part(20), _rows(20), _rows(1),
            _whole((20, 20)), _whole((20,)), _whole((20, 20)),
            _whole((20, 8)), _whole((8,)),
        ],
        out_specs=_rows(8),
        out_shape=jax.ShapeDtypeStruct((N_NODES, 8), jnp.float32),
    )(part3, h2, inv, wl_t, bl, wr_t, wc_t, bc)


def kernel(x, edge_index, Wl1, bl1, Wr1, Wl2, bl2, Wr2, Wl3, bl3, Wr3, Wc, bc):
    table1 = jnp.concatenate(
        [x, jnp.ones((N_NODES, 1), jnp.float32), jnp.zeros((N_NODES, 3), jnp.float32)],
        axis=1,
    )
    z8 = jnp.zeros((N_NODES, 8), jnp.float32)
    z16 = jnp.zeros((N_NODES, 16), jnp.float32)
    z20 = jnp.zeros((N_NODES, 20), jnp.float32)

    part1 = _agg8(table1, edge_index, z8)
    h1p, inv = _dense1(part1, x, Wl1.T, bl1, Wr1.T)

    part2 = _agg16(h1p, edge_index, z16)
    h2 = _dense2(part2, h1p, inv, Wl2.T, bl2, Wr2.T)

    part3 = _agg20(h2, edge_index, z20)
    return _dense3(part3, h2, inv, Wl3.T, bl3, Wr3.T, Wc.T, bc)


# speedup vs baseline: 14.0564x; 14.0564x over previous
import functools

import jax
import jax.numpy as jnp
from jax import lax
from jax.experimental import pallas as pl
from jax.experimental.pallas import tpu as pltpu, tpu_sc as plsc

N_NODES = 100000
N_EDGES = 6400000
NC = 2
NS = 16
NW = NC * NS
CH = 128
N_CHUNKS = N_EDGES // CH
BASE_CHUNKS = N_CHUNKS // NW
REM_CHUNKS = N_CHUNKS % NW
N_PAD = 100096
ROWS_PER_TILE = N_PAD // NS


def _make_agg(dp):
    mesh = plsc.VectorSubcoreMesh(
        core_axis_name="c", subcore_axis_name="s", num_cores=NC, num_subcores=NS
    )

    @functools.partial(
        pl.kernel,
        out_type=jax.ShapeDtypeStruct((NC, N_PAD, dp), jnp.float32),
        mesh=mesh,
        scratch_types=[
            pltpu.VMEM((2, CH), jnp.int32),
            pltpu.VMEM((CH, dp), jnp.float32),
            pltpu.VMEM_SHARED((N_PAD, dp), jnp.float32),
        ],
        compiler_params=pltpu.CompilerParams(use_tc_tiling_on_sc=False),
    )
    def agg_kernel(table, edges, zeros, out, idx_v, rows_v, acc):
        cid = lax.axis_index("c")
        sid = lax.axis_index("s")
        wid = sid * NC + cid

        r0 = sid * ROWS_PER_TILE
        pltpu.sync_copy(
            zeros.at[pl.ds(r0, ROWS_PER_TILE)], acc.at[pl.ds(r0, ROWS_PER_TILE)]
        )
        plsc.subcore_barrier()

        n_chunks = jnp.where(wid < REM_CHUNKS, BASE_CHUNKS + 1, BASE_CHUNKS)

        @pl.loop(0, n_chunks)
        def _(j):
            start = (wid + j * NW) * CH
            pltpu.sync_copy(edges.at[:, pl.ds(start, CH)], idx_v)
            pltpu.sync_copy(table.at[idx_v.at[0]], rows_v)
            pltpu.sync_copy(rows_v, acc.at[idx_v.at[1]], add=True)

        plsc.subcore_barrier()
        pltpu.sync_copy(
            acc.at[pl.ds(r0, ROWS_PER_TILE)], out.at[cid, pl.ds(r0, ROWS_PER_TILE)]
        )

    return agg_kernel


def _make_agg3():
    mesh = plsc.VectorSubcoreMesh(
        core_axis_name="c", subcore_axis_name="s", num_cores=NC, num_subcores=NS
    )

    @functools.partial(
        pl.kernel,
        out_type=jax.ShapeDtypeStruct((NC, N_PAD, 16), jnp.float32),
        mesh=mesh,
        scratch_types=[
            pltpu.VMEM((2, CH), jnp.int32),
            pltpu.VMEM((CH, 16), jnp.float32),
            pltpu.VMEM_SHARED((N_PAD, 16), jnp.float32),
        ],
        compiler_params=pltpu.CompilerParams(use_tc_tiling_on_sc=False),
    )
    def agg3_kernel(tables, edges, zeros, out, idx_v, rows_v, acc):
        cid = lax.axis_index("c")
        sid = lax.axis_index("s")
        r0 = sid * ROWS_PER_TILE
        pltpu.sync_copy(
            zeros.at[pl.ds(r0, ROWS_PER_TILE)], acc.at[pl.ds(r0, ROWS_PER_TILE)]
        )
        plsc.subcore_barrier()

        @pl.loop(0, N_CHUNKS // NS)
        def _(j):
            start = (sid + j * NS) * CH
            pltpu.sync_copy(edges.at[:, pl.ds(start, CH)], idx_v)
            pltpu.sync_copy(tables.at[cid].at[idx_v.at[0]], rows_v)
            pltpu.sync_copy(rows_v, acc.at[idx_v.at[1]], add=True)

        plsc.subcore_barrier()
        pltpu.sync_copy(
            acc.at[pl.ds(r0, ROWS_PER_TILE)], out.at[cid, pl.ds(r0, ROWS_PER_TILE)]
        )

    return agg3_kernel


_agg8 = _make_agg(8)
_agg16 = _make_agg(16)
_agg3 = _make_agg3()

_BLK = 2000
_GRID = N_NODES // _BLK


def _l1_body(p_ref, x_ref, wl_ref, bl_ref, wr_ref, h_ref, inv_ref):
    p = p_ref[0] + p_ref[1]
    inv = 1.0 / jnp.maximum(p[:, 4:5], 1.0)
    mean = p[:, :4] * inv
    h = jnp.maximum(mean @ wl_ref[...] + bl_ref[...] + x_ref[...] @ wr_ref[...], 0.0)
    h_ref[...] = jnp.concatenate([h, jnp.zeros((_BLK, 6), jnp.float32)], axis=1)
    inv_ref[...] = inv


def _l2_body(p_ref, x_ref, inv_ref, wl_ref, bl_ref, wr_ref, h_ref):
    p = p_ref[0] + p_ref[1]
    mean = p[:, :10] * inv_ref[...]
    x10 = x_ref[...][:, :10]
    h = jnp.maximum(mean @ wl_ref[...] + bl_ref[...] + x10 @ wr_ref[...], 0.0)
    z6 = jnp.zeros((_BLK, 6), jnp.float32)
    h_ref[...] = jnp.stack(
        [jnp.concatenate([h[:, :10], z6], axis=1),
         jnp.concatenate([h[:, 10:], z6], axis=1)],
        axis=0,
    )


def _l3_body(p_ref, x_ref, inv_ref, wl_ref, bl_ref, wr_ref, wc_ref, bc_ref, o_ref):
    p = p_ref[...]
    mean = jnp.concatenate([p[0, :, :10], p[1, :, :10]], axis=1) * inv_ref[...]
    x20 = jnp.concatenate([x_ref[0, :, :10], x_ref[1, :, :10]], axis=1)
    h = jnp.maximum(mean @ wl_ref[...] + bl_ref[...] + x20 @ wr_ref[...], 0.0)
    o_ref[...] = h @ wc_ref[...] + bc_ref[...]


def _whole(shape):
    return pl.BlockSpec(shape, lambda i: (0,) * len(shape))


def _rows(d):
    return pl.BlockSpec((_BLK, d), lambda i: (i, 0))


def _part(dp):
    return pl.BlockSpec((2, _BLK, dp), lambda i: (0, i, 0))


def _dense1(part1, x, wl_t, bl, wr_t):
    return pl.pallas_call(
        _l1_body,
        grid=(_GRID,),
        in_specs=[_part(8), _rows(4), _whole((4, 10)), _whole((10,)), _whole((4, 10))],
        out_specs=[_rows(16), _rows(1)],
        out_shape=[
            jax.ShapeDtypeStruct((N_NODES, 16), jnp.float32),
            jax.ShapeDtypeStruct((N_NODES, 1), jnp.float32),
        ],
    )(part1, x, wl_t, bl, wr_t)


def _dense2(part2, h1p, inv, wl_t, bl, wr_t):
    return pl.pallas_call(
        _l2_body,
        grid=(_GRID,),
        in_specs=[
            _part(16), _rows(16), _rows(1),
            _whole((10, 20)), _whole((20,)), _whole((10, 20)),
        ],
        out_specs=pl.BlockSpec((2, _BLK, 16), lambda i: (0, i, 0)),
        out_shape=jax.ShapeDtypeStruct((2, N_NODES, 16), jnp.float32),
    )(part2, h1p, inv, wl_t, bl, wr_t)


def _dense3(part3, h2s, inv, wl_t, bl, wr_t, wc_t, bc):
    return pl.pallas_call(
        _l3_body,
        grid=(_GRID,),
        in_specs=[
            _part(16), pl.BlockSpec((2, _BLK, 16), lambda i: (0, i, 0)), _rows(1),
            _whole((20, 20)), _whole((20,)), _whole((20, 20)),
            _whole((20, 8)), _whole((8,)),
        ],
        out_specs=_rows(8),
        out_shape=jax.ShapeDtypeStruct((N_NODES, 8), jnp.float32),
    )(part3, h2s, inv, wl_t, bl, wr_t, wc_t, bc)


def kernel(x, edge_index, Wl1, bl1, Wr1, Wl2, bl2, Wr2, Wl3, bl3, Wr3, Wc, bc):
    table1 = jnp.concatenate(
        [x, jnp.ones((N_NODES, 1), jnp.float32), jnp.zeros((N_NODES, 3), jnp.float32)],
        axis=1,
    )
    z8 = jnp.zeros((N_PAD, 8), jnp.float32)
    z16 = jnp.zeros((N_PAD, 16), jnp.float32)

    part1 = _agg8(table1, edge_index, z8)
    h1p, inv = _dense1(part1, x, Wl1.T, bl1, Wr1.T)

    part2 = _agg16(h1p, edge_index, z16)
    h2s = _dense2(part2, h1p, inv, Wl2.T, bl2, Wr2.T)

    part3 = _agg3(h2s, edge_index, z16)
    return _dense3(part3, h2s, inv, Wl3.T, bl3, Wr3.T, Wc.T, bc)


# speedup vs baseline: 41.7165x; 2.9678x over previous
import functools

import jax
import jax.numpy as jnp
from jax import lax
from jax.experimental import pallas as pl
from jax.experimental.pallas import tpu as pltpu, tpu_sc as plsc

N_NODES = 100000
N_EDGES = 6400000
NC = 2
NS = 16
NW = NC * NS
CH = 128
N_CHUNKS = N_EDGES // CH
BASE_CHUNKS = N_CHUNKS // NW
REM_CHUNKS = N_CHUNKS % NW
N_PAD = 100096
ROWS_PER_TILE = N_PAD // NS


K_BLK = 8
N_BLOCKS = N_CHUNKS // K_BLK


def _make_agg(dp):
    mesh = plsc.VectorSubcoreMesh(
        core_axis_name="c", subcore_axis_name="s", num_cores=NC, num_subcores=NS
    )
    base_blocks = N_BLOCKS // NW
    rem_blocks = N_BLOCKS % NW

    @functools.partial(
        pl.kernel,
        out_type=jax.ShapeDtypeStruct((NC, N_PAD, dp), jnp.float32),
        mesh=mesh,
        scratch_types=[
            pltpu.VMEM((K_BLK, CH), jnp.int32),
            pltpu.VMEM((K_BLK, CH), jnp.int32),
            pltpu.VMEM((K_BLK, CH, dp), jnp.float32),
            pltpu.VMEM_SHARED((N_PAD, dp), jnp.float32),
            pltpu.SemaphoreType.DMA,
            pltpu.SemaphoreType.DMA,
            pltpu.SemaphoreType.DMA,
        ],
        compiler_params=pltpu.CompilerParams(use_tc_tiling_on_sc=False),
    )
    def agg_kernel(table, edges, zeros, out, src_v, dst_v, rows_v, acc,
                   sem_i, sem_g, sem_s):
        cid = lax.axis_index("c")
        sid = lax.axis_index("s")
        wid = sid * NC + cid

        r0 = sid * ROWS_PER_TILE
        pltpu.sync_copy(
            zeros.at[pl.ds(r0, ROWS_PER_TILE)], acc.at[pl.ds(r0, ROWS_PER_TILE)]
        )
        plsc.subcore_barrier()

        n_blocks = jnp.where(wid < rem_blocks, base_blocks + 1, base_blocks)

        @pl.loop(0, n_blocks)
        def _(m):
            c0 = (wid + m * NW) * K_BLK
            di = pltpu.async_copy(edges.at[0, pl.ds(c0, K_BLK)], src_v, sem_i)
            dj = pltpu.async_copy(edges.at[1, pl.ds(c0, K_BLK)], dst_v, sem_i)
            di.wait()
            dj.wait()
            gs = [
                pltpu.async_copy(table.at[src_v.at[k]], rows_v.at[k], sem_g)
                for k in range(K_BLK)
            ]
            for g in gs:
                g.wait()
            ss = [
                pltpu.async_copy(rows_v.at[k], acc.at[dst_v.at[k]], sem_s, add=True)
                for k in range(K_BLK)
            ]
            for sdesc in ss:
                sdesc.wait()

        plsc.subcore_barrier()
        pltpu.sync_copy(
            acc.at[pl.ds(r0, ROWS_PER_TILE)], out.at[cid, pl.ds(r0, ROWS_PER_TILE)]
        )

    return agg_kernel


def _make_agg3():
    mesh = plsc.VectorSubcoreMesh(
        core_axis_name="c", subcore_axis_name="s", num_cores=NC, num_subcores=NS
    )
    base_blocks = N_BLOCKS // NS
    rem_blocks = N_BLOCKS % NS

    @functools.partial(
        pl.kernel,
        out_type=jax.ShapeDtypeStruct((NC, N_PAD, 16), jnp.float32),
        mesh=mesh,
        scratch_types=[
            pltpu.VMEM((K_BLK, CH), jnp.int32),
            pltpu.VMEM((K_BLK, CH), jnp.int32),
            pltpu.VMEM((K_BLK, CH, 16), jnp.float32),
            pltpu.VMEM_SHARED((N_PAD, 16), jnp.float32),
            pltpu.SemaphoreType.DMA,
            pltpu.SemaphoreType.DMA,
            pltpu.SemaphoreType.DMA,
        ],
        compiler_params=pltpu.CompilerParams(use_tc_tiling_on_sc=False),
    )
    def agg3_kernel(tables, edges, zeros, out, src_v, dst_v, rows_v, acc,
                    sem_i, sem_g, sem_s):
        cid = lax.axis_index("c")
        sid = lax.axis_index("s")
        r0 = sid * ROWS_PER_TILE
        pltpu.sync_copy(
            zeros.at[pl.ds(r0, ROWS_PER_TILE)], acc.at[pl.ds(r0, ROWS_PER_TILE)]
        )
        plsc.subcore_barrier()

        n_blocks = jnp.where(sid < rem_blocks, base_blocks + 1, base_blocks)

        @pl.loop(0, n_blocks)
        def _(m):
            c0 = (sid + m * NS) * K_BLK
            di = pltpu.async_copy(edges.at[0, pl.ds(c0, K_BLK)], src_v, sem_i)
            dj = pltpu.async_copy(edges.at[1, pl.ds(c0, K_BLK)], dst_v, sem_i)
            di.wait()
            dj.wait()
            gs = [
                pltpu.async_copy(
                    tables.at[cid].at[src_v.at[k]], rows_v.at[k], sem_g
                )
                for k in range(K_BLK)
            ]
            for g in gs:
                g.wait()
            ss = [
                pltpu.async_copy(rows_v.at[k], acc.at[dst_v.at[k]], sem_s, add=True)
                for k in range(K_BLK)
            ]
            for sdesc in ss:
                sdesc.wait()

        plsc.subcore_barrier()
        pltpu.sync_copy(
            acc.at[pl.ds(r0, ROWS_PER_TILE)], out.at[cid, pl.ds(r0, ROWS_PER_TILE)]
        )

    return agg3_kernel


_agg8 = _make_agg(8)
_agg16 = _make_agg(16)
_agg3 = _make_agg3()

_BLK = 2000
_GRID = N_NODES // _BLK


def _l1_body(p_ref, x_ref, wl_ref, bl_ref, wr_ref, h_ref, inv_ref):
    p = p_ref[0] + p_ref[1]
    inv = 1.0 / jnp.maximum(p[:, 4:5], 1.0)
    mean = p[:, :4] * inv
    h = jnp.maximum(mean @ wl_ref[...] + bl_ref[...] + x_ref[...] @ wr_ref[...], 0.0)
    h_ref[...] = jnp.concatenate([h, jnp.zeros((_BLK, 6), jnp.float32)], axis=1)
    inv_ref[...] = inv


def _l2_body(p_ref, x_ref, inv_ref, wl_ref, bl_ref, wr_ref, h_ref):
    p = p_ref[0] + p_ref[1]
    mean = p[:, :10] * inv_ref[...]
    x10 = x_ref[...][:, :10]
    h = jnp.maximum(mean @ wl_ref[...] + bl_ref[...] + x10 @ wr_ref[...], 0.0)
    z6 = jnp.zeros((_BLK, 6), jnp.float32)
    h_ref[...] = jnp.stack(
        [jnp.concatenate([h[:, :10], z6], axis=1),
         jnp.concatenate([h[:, 10:], z6], axis=1)],
        axis=0,
    )


def _l3_body(p_ref, x_ref, inv_ref, wl_ref, bl_ref, wr_ref, wc_ref, bc_ref, o_ref):
    p = p_ref[...]
    mean = jnp.concatenate([p[0, :, :10], p[1, :, :10]], axis=1) * inv_ref[...]
    x20 = jnp.concatenate([x_ref[0, :, :10], x_ref[1, :, :10]], axis=1)
    h = jnp.maximum(mean @ wl_ref[...] + bl_ref[...] + x20 @ wr_ref[...], 0.0)
    o_ref[...] = h @ wc_ref[...] + bc_ref[...]


def _whole(shape):
    return pl.BlockSpec(shape, lambda i: (0,) * len(shape))


def _rows(d):
    return pl.BlockSpec((_BLK, d), lambda i: (i, 0))


def _part(dp):
    return pl.BlockSpec((2, _BLK, dp), lambda i: (0, i, 0))


def _dense1(part1, x, wl_t, bl, wr_t):
    return pl.pallas_call(
        _l1_body,
        grid=(_GRID,),
        in_specs=[_part(8), _rows(4), _whole((4, 10)), _whole((10,)), _whole((4, 10))],
        out_specs=[_rows(16), _rows(1)],
        out_shape=[
            jax.ShapeDtypeStruct((N_NODES, 16), jnp.float32),
            jax.ShapeDtypeStruct((N_NODES, 1), jnp.float32),
        ],
    )(part1, x, wl_t, bl, wr_t)


def _dense2(part2, h1p, inv, wl_t, bl, wr_t):
    return pl.pallas_call(
        _l2_body,
        grid=(_GRID,),
        in_specs=[
            _part(16), _rows(16), _rows(1),
            _whole((10, 20)), _whole((20,)), _whole((10, 20)),
        ],
        out_specs=pl.BlockSpec((2, _BLK, 16), lambda i: (0, i, 0)),
        out_shape=jax.ShapeDtypeStruct((2, N_NODES, 16), jnp.float32),
    )(part2, h1p, inv, wl_t, bl, wr_t)


def _dense3(part3, h2s, inv, wl_t, bl, wr_t, wc_t, bc):
    return pl.pallas_call(
        _l3_body,
        grid=(_GRID,),
        in_specs=[
            _part(16), pl.BlockSpec((2, _BLK, 16), lambda i: (0, i, 0)), _rows(1),
            _whole((20, 20)), _whole((20,)), _whole((20, 20)),
            _whole((20, 8)), _whole((8,)),
        ],
        out_specs=_rows(8),
        out_shape=jax.ShapeDtypeStruct((N_NODES, 8), jnp.float32),
    )(part3, h2s, inv, wl_t, bl, wr_t, wc_t, bc)


def kernel(x, edge_index, Wl1, bl1, Wr1, Wl2, bl2, Wr2, Wl3, bl3, Wr3, Wc, bc):
    edges3 = edge_index.reshape(2, N_CHUNKS, CH)
    table1 = jnp.concatenate(
        [x, jnp.ones((N_NODES, 1), jnp.float32), jnp.zeros((N_NODES, 3), jnp.float32)],
        axis=1,
    )
    z8 = jnp.zeros((N_PAD, 8), jnp.float32)
    z16 = jnp.zeros((N_PAD, 16), jnp.float32)

    part1 = _agg8(table1, edges3, z8)
    h1p, inv = _dense1(part1, x, Wl1.T, bl1, Wr1.T)

    part2 = _agg16(h1p, edges3, z16)
    h2s = _dense2(part2, h1p, inv, Wl2.T, bl2, Wr2.T)

    part3 = _agg3(h2s, edges3, z16)
    return _dense3(part3, h2s, inv, Wl3.T, bl3, Wr3.T, Wc.T, bc)
